# baseline (speedup 1.0000x reference)
import functools

import jax
import jax.numpy as jnp
from jax import lax
from jax.experimental import pallas as pl
from jax.experimental.pallas import tpu as pltpu
from jax.experimental.pallas import tpu_sc as plsc

NC = 2
NS = 16
NW = NC * NS

S = 50
NB = 16384
B = S * NB
D = 64
BLK = 128
NBLK = B // BLK
BPW = B // NW
GPW = BPW // BLK
JB = NB // BLK

_mesh = plsc.VectorSubcoreMesh(
    core_axis_name="c", subcore_axis_name="s", num_cores=NC, num_subcores=NS
)


@functools.partial(
    pl.kernel,
    out_type=jax.ShapeDtypeStruct((S, D // 8, JB, 8, BLK), jnp.float32),
    mesh=_mesh,
    compiler_params=pltpu.CompilerParams(
        use_tc_tiling_on_sc=False, needs_layout_passes=False),
    scratch_types=[
        pltpu.VMEM((BPW,), jnp.int32),
        pltpu.VMEM((BLK, D), jnp.float32),
        pltpu.VMEM((BLK, D), jnp.float32),
        pltpu.VMEM((D, BLK), jnp.float32),
        pltpu.VMEM((D, BLK), jnp.float32),
        pltpu.SemaphoreType.DMA,
        pltpu.SemaphoreType.DMA,
        pltpu.SemaphoreType.DMA,
        pltpu.SemaphoreType.DMA,
    ],
)
def _embed_gather(idx_hbm, table_hbm, out_hbm,
                  idx_v, gbuf0, gbuf1, tbuf0, tbuf1, g0, g1, o0, o1):
    wid = lax.axis_index("s") * NC + lax.axis_index("c")
    base = wid * BPW
    blk0 = wid * GPW

    pltpu.sync_copy(idx_hbm.at[pl.ds(base, BPW)], idx_v)

    iota = lax.iota(jnp.int32, 16)
    drows = [16 * dg + iota for dg in range(4)]

    def fire(g, buf, sem):
        off = pl.multiple_of(g * BLK, BLK)
        pltpu.async_copy(table_hbm.at[idx_v.at[pl.ds(off, BLK)]], buf, sem)

    def drain_g(buf, sem):
        pltpu.make_async_copy(table_hbm.at[pl.ds(0, BLK)], buf, sem).wait()

    def transpose(gbuf, tbuf):
        for b in range(BLK):
            col = jnp.full((16,), b, jnp.int32)
            for dg in range(4):
                v = gbuf[b, pl.ds(16 * dg, 16)]
                plsc.store_scatter(tbuf, [drows[dg], col], v)

    def write(blk, tbuf, sem):
        s = blk // JB
        j = lax.rem(blk, JB)
        for gr in range(D // 8):
            pltpu.async_copy(
                tbuf.at[pl.ds(8 * gr, 8)], out_hbm.at[s, gr, j], sem)

    def drain_o(tbuf, sem):
        for gr in range(D // 8):
            pltpu.make_async_copy(
                out_hbm.at[0, 0, 0], tbuf.at[pl.ds(8 * gr, 8)], sem).wait()

    fire(0, gbuf0, g0)
    fire(1, gbuf1, g1)
    write(blk0, tbuf0, o0)
    write(blk0 + 1, tbuf1, o1)

    def body(i, _):
        ga = 2 * i
        drain_g(gbuf0, g0)
        drain_o(tbuf0, o0)
        transpose(gbuf0, tbuf0)
        fire(jnp.minimum(ga + 2, GPW - 1), gbuf0, g0)
        write(blk0 + ga, tbuf0, o0)
        drain_g(gbuf1, g1)
        drain_o(tbuf1, o1)
        transpose(gbuf1, tbuf1)
        fire(jnp.minimum(ga + 3, GPW - 1), gbuf1, g1)
        write(blk0 + ga + 1, tbuf1, o1)
        return 0

    lax.fori_loop(0, GPW // 2, body, 0)
    drain_g(gbuf0, g0)
    drain_g(gbuf1, g1)
    drain_o(tbuf0, o0)
    drain_o(tbuf1, o1)


def kernel(x, table):
    flat = x.T.reshape(-1).astype(jnp.int32)
    out5 = _embed_gather(flat, table)
    return out5.transpose(2, 4, 0, 1, 3).reshape(NB, S, D)
